# baseline (speedup 1.0000x reference)
import functools

import jax
import jax.numpy as jnp
from jax import lax
from jax.experimental import pallas as pl
from jax.experimental.pallas import tpu as pltpu
from jax.experimental.pallas import tpu_sc as plsc

NC, NS, L = 2, 16, 16
NW = NC * NS
N_POINTS = 262144
C = 32
NP = 9
B = 128
CHUNKS = N_POINTS // (NW * B)
RESS = [128, 256, 512] * 3

_mesh = plsc.VectorSubcoreMesh(
    core_axis_name="c", subcore_axis_name="s", num_cores=NC, num_subcores=NS
)


@functools.partial(
    pl.kernel,
    out_type=jax.ShapeDtypeStruct((N_POINTS, NP * C), jnp.float32),
    mesh=_mesh,
    compiler_params=pltpu.CompilerParams(
        needs_layout_passes=False, use_tc_tiling_on_sc=False
    ),
    scratch_types=[
        pltpu.VMEM((3, B), jnp.float32),
        pltpu.VMEM((2, 2, B), jnp.int32),
        pltpu.VMEM((2, 2, B), jnp.float32),
        pltpu.VMEM((2, 2, B, 2 * C), jnp.bfloat16),
        pltpu.VMEM((B, NP * C), jnp.float32),
        pltpu.SemaphoreType.DMA,
        pltpu.SemaphoreType.DMA,
    ],
)
def _encode(x0_h, x1_h, x2_h, t0, t1, t2, t3, t4, t5, t6, t7, t8, out_h,
            xv, idxv, wv, rowsv, outv, sem0, sem1):
    wid = lax.axis_index("s") * NC + lax.axis_index("c")
    tables = [t0, t1, t2, t3, t4, t5, t6, t7, t8]
    xs = [x0_h, x1_h, x2_h]
    sems = [sem0, sem1]

    def chunk_body(ci, carry):
        base = (wid * CHUNKS + ci) * B
        for d in range(3):
            pltpu.sync_copy(xs[d].at[pl.ds(base, B)], xv.at[d])

        def stage(k):
            par = k % 2
            res = RESS[k]
            gdim = k // 3
            for g in range(B // L):
                s = pl.ds(g * L, L)
                gx = xv[gdim, s]
                gy = xv[(gdim + 1) % 3, s]
                fres = float(res - 1)
                cx = (gx + 1.0) * fres * 0.5
                cy = (gy + 1.0) * fres * 0.5
                ix = jnp.clip((cx + 1.0) * 0.5 * fres, 0.0, fres)
                iy = jnp.clip((cy + 1.0) * 0.5 * fres, 0.0, fres)
                x0 = ix.astype(jnp.int32)
                y0 = iy.astype(jnp.int32)
                wv[par, 0, s] = ix - x0.astype(jnp.float32)
                wv[par, 1, s] = iy - y0.astype(jnp.float32)
                y1 = jnp.minimum(y0 + 1, res - 1)
                idxv[par, 0, s] = y0 * res + x0
                idxv[par, 1, s] = y1 * res + x0
            tbl = tables[k]
            return [
                pltpu.async_copy(
                    tbl.at[idxv.at[par, q]], rowsv.at[par, q], sems[par]
                )
                for q in range(2)
            ]

        def combine(k):
            par = k % 2

            @plsc.parallel_loop(0, B, 1, unroll=8)
            def comb_body(p, k=k, par=par):
                pv = jnp.full((L,), p, jnp.int32)
                wx = plsc.load_gather(wv.at[par, 0], [pv])
                wy = plsc.load_gather(wv.at[par, 1], [pv])
                wxb = plsc.pack(wx, wx, format=plsc.PackFormat.INTERLEAVED)
                wyb = plsc.pack(wy, wy, format=plsc.PackFormat.INTERLEAVED)
                v00 = rowsv[par, 0, p, pl.ds(0, C)]
                v01 = rowsv[par, 0, p, pl.ds(C, C)]
                v10 = rowsv[par, 1, p, pl.ds(0, C)]
                v11 = rowsv[par, 1, p, pl.ds(C, C)]
                top = v00 + wxb * (v01 - v00)
                bot = v10 + wxb * (v11 - v10)
                res = top + wyb * (bot - top)
                lo, hi = plsc.unpack(res, format=plsc.PackFormat.INTERLEAVED)
                outv[p, pl.ds(k * C, L)] = lo
                outv[p, pl.ds(k * C + L, L)] = hi

        cps = stage(0)
        for k in range(NP):
            nxt = stage(k + 1) if k + 1 < NP else None
            for cp in cps:
                cp.wait()
            combine(k)
            cps = nxt

        pltpu.sync_copy(outv, out_h.at[pl.ds(base, B)])
        return carry

    lax.fori_loop(0, CHUNKS, chunk_body, 0)


def kernel(x, plane_0, plane_1, plane_2, plane_3, plane_4, plane_5, plane_6,
           plane_7, plane_8):
    planes = (plane_0, plane_1, plane_2, plane_3, plane_4, plane_5, plane_6,
              plane_7, plane_8)
    tables = []
    for p in planes:
        t = (p[0].reshape(2, C // 2, -1).transpose(2, 1, 0)
             .reshape(-1, C).astype(jnp.bfloat16))
        tshift = jnp.concatenate(
            [t[1:], jnp.zeros((1, C), jnp.bfloat16)], axis=0)
        tables.append(jnp.concatenate([t, tshift], axis=1))
    return _encode(x[:, 0], x[:, 1], x[:, 2], *tables)


# speedup vs baseline: 1.1198x; 1.1198x over previous
import functools

import jax
import jax.numpy as jnp
from jax import lax
from jax.experimental import pallas as pl
from jax.experimental.pallas import tpu as pltpu
from jax.experimental.pallas import tpu_sc as plsc

NC, NS, L = 2, 16, 16
NW = NC * NS
N_POINTS = 262144
C = 32
NP = 9
B = 128
CHUNKS = N_POINTS // (NW * B)
RESS = [128, 256, 512] * 3

_mesh = plsc.VectorSubcoreMesh(
    core_axis_name="c", subcore_axis_name="s", num_cores=NC, num_subcores=NS
)


@functools.partial(
    pl.kernel,
    out_type=jax.ShapeDtypeStruct((N_POINTS, NP * C), jnp.float32),
    mesh=_mesh,
    compiler_params=pltpu.CompilerParams(
        needs_layout_passes=False, use_tc_tiling_on_sc=False
    ),
    scratch_types=[
        pltpu.VMEM((3, B), jnp.float32),
        pltpu.VMEM((2, 4, B), jnp.int32),
        pltpu.VMEM((2, 2, B), jnp.float32),
        pltpu.VMEM((2, 4, B, C), jnp.bfloat16),
        pltpu.VMEM((B, NP * C), jnp.float32),
        pltpu.SemaphoreType.DMA,
        pltpu.SemaphoreType.DMA,
    ],
)
def _encode(x0_h, x1_h, x2_h, t0, t1, t2, t3, t4, t5, t6, t7, t8, out_h,
            xv, idxv, wv, rowsv, outv, sem0, sem1):
    wid = lax.axis_index("s") * NC + lax.axis_index("c")
    tables = [t0, t1, t2, t3, t4, t5, t6, t7, t8]
    xs = [x0_h, x1_h, x2_h]
    sems = [sem0, sem1]

    def chunk_body(ci, carry):
        base = (wid * CHUNKS + ci) * B
        for d in range(3):
            pltpu.sync_copy(xs[d].at[pl.ds(base, B)], xv.at[d])

        def stage(k):
            par = k % 2
            res = RESS[k]
            gdim = k // 3
            for g in range(B // L):
                s = pl.ds(g * L, L)
                gx = xv[gdim, s]
                gy = xv[(gdim + 1) % 3, s]
                fres = float(res - 1)
                cx = (gx + 1.0) * fres * 0.5
                cy = (gy + 1.0) * fres * 0.5
                ix = jnp.clip((cx + 1.0) * 0.5 * fres, 0.0, fres)
                iy = jnp.clip((cy + 1.0) * 0.5 * fres, 0.0, fres)
                x0 = ix.astype(jnp.int32)
                y0 = iy.astype(jnp.int32)
                wv[par, 0, s] = ix - x0.astype(jnp.float32)
                wv[par, 1, s] = iy - y0.astype(jnp.float32)
                x1 = jnp.minimum(x0 + 1, res - 1)
                y1 = jnp.minimum(y0 + 1, res - 1)
                rowb = y0 * res
                rowt = y1 * res
                idxv[par, 0, s] = rowb + x0
                idxv[par, 1, s] = rowb + x1
                idxv[par, 2, s] = rowt + x0
                idxv[par, 3, s] = rowt + x1
            tbl = tables[k]
            return [
                pltpu.async_copy(
                    tbl.at[idxv.at[par, q]], rowsv.at[par, q], sems[par]
                )
                for q in range(4)
            ]

        def combine(k):
            par = k % 2

            @plsc.parallel_loop(0, B, 1, unroll=8)
            def comb_body(p, k=k, par=par):
                pv = jnp.full((L,), p, jnp.int32)
                wx = plsc.load_gather(wv.at[par, 0], [pv])
                wy = plsc.load_gather(wv.at[par, 1], [pv])
                wxb = plsc.pack(wx, wx, format=plsc.PackFormat.INTERLEAVED)
                wyb = plsc.pack(wy, wy, format=plsc.PackFormat.INTERLEAVED)
                v00 = rowsv[par, 0, p, :]
                v01 = rowsv[par, 1, p, :]
                v10 = rowsv[par, 2, p, :]
                v11 = rowsv[par, 3, p, :]
                top = v00 + wxb * (v01 - v00)
                bot = v10 + wxb * (v11 - v10)
                res = top + wyb * (bot - top)
                lo, hi = plsc.unpack(res, format=plsc.PackFormat.INTERLEAVED)
                outv[p, pl.ds(k * C, L)] = lo
                outv[p, pl.ds(k * C + L, L)] = hi

        cps = stage(0)
        for k in range(NP):
            nxt = stage(k + 1) if k + 1 < NP else None
            for cp in cps:
                cp.wait()
            combine(k)
            cps = nxt

        pltpu.sync_copy(outv, out_h.at[pl.ds(base, B)])
        return carry

    lax.fori_loop(0, CHUNKS, chunk_body, 0)


def kernel(x, plane_0, plane_1, plane_2, plane_3, plane_4, plane_5, plane_6,
           plane_7, plane_8):
    planes = (plane_0, plane_1, plane_2, plane_3, plane_4, plane_5, plane_6,
              plane_7, plane_8)
    tables = [
        p[0].reshape(2, C // 2, -1).transpose(2, 1, 0)
        .reshape(-1, C).astype(jnp.bfloat16)
        for p in planes
    ]
    return _encode(x[:, 0], x[:, 1], x[:, 2], *tables)


# speedup vs baseline: 1.1812x; 1.0548x over previous
import functools

import jax
import jax.numpy as jnp
from jax import lax
from jax.experimental import pallas as pl
from jax.experimental.pallas import tpu as pltpu
from jax.experimental.pallas import tpu_sc as plsc

NC, NS, L = 2, 16, 16
NW = NC * NS
N_POINTS = 262144
C = 32
NP = 9
B = 128
CHUNKS = N_POINTS // (NW * B)
RESS = [128, 256, 512] * 3

_mesh = plsc.VectorSubcoreMesh(
    core_axis_name="c", subcore_axis_name="s", num_cores=NC, num_subcores=NS
)


@functools.partial(
    pl.kernel,
    out_type=jax.ShapeDtypeStruct((N_POINTS, NP * C), jnp.float32),
    mesh=_mesh,
    compiler_params=pltpu.CompilerParams(
        needs_layout_passes=False, use_tc_tiling_on_sc=False
    ),
    scratch_types=[
        pltpu.VMEM((3, B), jnp.float32),
        pltpu.VMEM((2, 4, B), jnp.int32),
        pltpu.VMEM((2, 2, B), jnp.float32),
        pltpu.VMEM((2, 4, B, C), jnp.bfloat16),
        pltpu.VMEM((B, NP * C), jnp.float32),
        pltpu.SemaphoreType.DMA,
        pltpu.SemaphoreType.DMA,
    ],
)
def _encode(x0_h, x1_h, x2_h, t0, t1, t2, t3, t4, t5, t6, t7, t8, out_h,
            xv, idxv, wv, rowsv, outv, sem0, sem1):
    wid = lax.axis_index("s") * NC + lax.axis_index("c")
    tables = [t0, t1, t2, t3, t4, t5, t6, t7, t8]
    xs = [x0_h, x1_h, x2_h]
    sems = [sem0, sem1]

    def chunk_body(ci, carry):
        base = (wid * CHUNKS + ci) * B
        for d in range(3):
            pltpu.sync_copy(xs[d].at[pl.ds(base, B)], xv.at[d])

        def stage(k):
            par = k % 2
            res = RESS[k]
            gdim = k // 3
            for g in range(B // L):
                s = pl.ds(g * L, L)
                gx = xv[gdim, s]
                gy = xv[(gdim + 1) % 3, s]
                fres = float(res - 1)
                cx = (gx + 1.0) * fres * 0.5
                cy = (gy + 1.0) * fres * 0.5
                ix = jnp.clip((cx + 1.0) * 0.5 * fres, 0.0, fres)
                iy = jnp.clip((cy + 1.0) * 0.5 * fres, 0.0, fres)
                x0 = ix.astype(jnp.int32)
                y0 = iy.astype(jnp.int32)
                wv[par, 0, s] = ix - x0.astype(jnp.float32)
                wv[par, 1, s] = iy - y0.astype(jnp.float32)
                x1 = jnp.minimum(x0 + 1, res - 1)
                y1 = jnp.minimum(y0 + 1, res - 1)
                rowb = y0 * res
                rowt = y1 * res
                idxv[par, 0, s] = rowb + x0
                idxv[par, 1, s] = rowb + x1
                idxv[par, 2, s] = rowt + x0
                idxv[par, 3, s] = rowt + x1
            tbl = tables[k]
            return [
                pltpu.async_copy(
                    tbl.at[idxv.at[par, q]], rowsv.at[par, q], sems[par]
                )
                for q in range(4)
            ]

        def combine(k):
            par = k % 2

            @plsc.parallel_loop(0, B, 1, unroll=8)
            def comb_body(p, k=k, par=par):
                pv = jnp.full((L,), p, jnp.int32)
                wx = plsc.load_gather(wv.at[par, 0], [pv])
                wy = plsc.load_gather(wv.at[par, 1], [pv])
                wxb = plsc.pack(wx, wx, format=plsc.PackFormat.INTERLEAVED)
                wyb = plsc.pack(wy, wy, format=plsc.PackFormat.INTERLEAVED)
                v00 = rowsv[par, 0, p, :]
                v01 = rowsv[par, 1, p, :]
                v10 = rowsv[par, 2, p, :]
                v11 = rowsv[par, 3, p, :]
                top = v00 + wxb * (v01 - v00)
                bot = v10 + wxb * (v11 - v10)
                res = top + wyb * (bot - top)
                lo, hi = plsc.unpack(res, format=plsc.PackFormat.INTERLEAVED)
                outv[p, pl.ds(k * C, L)] = lo
                outv[p, pl.ds(k * C + L, L)] = hi

        cps = stage(0)
        for k in range(NP):
            nxt = stage(k + 1) if k + 1 < NP else None
            for cp in cps:
                cp.wait()
            combine(k)
            cps = nxt

        pltpu.sync_copy(outv, out_h.at[pl.ds(base, B)])
        return carry

    lax.fori_loop(0, CHUNKS, chunk_body, 0)


def kernel(x, plane_0, plane_1, plane_2, plane_3, plane_4, plane_5, plane_6,
           plane_7, plane_8):
    planes = (plane_0, plane_1, plane_2, plane_3, plane_4, plane_5, plane_6,
              plane_7, plane_8)
    perm = jnp.arange(C).reshape(2, C // 2).T.reshape(-1)
    tables = [p[0][perm].reshape(C, -1).T.astype(jnp.bfloat16)
              for p in planes]
    return _encode(x[:, 0], x[:, 1], x[:, 2], *tables)
